# baseline (speedup 1.0000x reference)
import functools

import jax
import jax.numpy as jnp
from jax.experimental import pallas as pl
from jax.experimental.pallas import tpu as pltpu

B = 512
D_Q = 1024
D_Z = 512
D_RIN = 256
D_PHI = 128
D_R = 9
M = 512
P = 256
NPROG = 8192
LAM = 0.1
H = 96

TB = 64
TPROG = 1024
NBUF = 8
NT = NPROG // TPROG


def _gelu_erf(x):
    halfx = x * jnp.asarray(0.5, x.dtype)
    return halfx * jax.lax.erf(x * jnp.asarray(0.7071067811865476, x.dtype)) + halfx


def _dotnn(a, b):
    return jax.lax.dot_general(a, b, (((1,), (0,)), ((), ())),
                               preferred_element_type=jnp.float32)


def _dotnt(a, b):
    return jax.lax.dot_general(a, b, (((1,), (1,)), ((), ())),
                               preferred_element_type=jnp.float32)


def _router_kernel(q_ref, r_ref, pair_idx_ref, relf_ref,
                   Wq1_ref, bq1_ref, Wq2_ref, bq2_ref,
                   We_ref, be_ref, Wu_ref,
                   W1z_ref, W1s_ref, W1a_ref, W1m_ref, W1r_ref, b1_ref,
                   W2_ref, b2_ref, W3_ref, b3_ref, len_ref,
                   A_hbm, Bp_hbm,
                   out_hbm,
                   u_s, v_s, A_buf, Bp_buf, out_buf, in_sem, out_sem):
    bf = jnp.bfloat16

    def a_copy(t):
        slot = t % NBUF
        return pltpu.make_async_copy(
            A_hbm.at[pl.ds(t * TPROG, TPROG), :], A_buf.at[slot], in_sem.at[slot])

    def b_copy(t):
        slot = t % NBUF
        return pltpu.make_async_copy(
            Bp_hbm.at[pl.ds(t * TPROG, TPROG), :], Bp_buf.at[slot], in_sem.at[slot])

    def o_copy(t):
        oslot = t % 2
        return pltpu.make_async_copy(
            out_buf.at[oslot], out_hbm.at[:, pl.ds(t * TPROG, TPROG)], out_sem.at[oslot])

    for t in range(min(NBUF, NT)):
        a_copy(t).start()
        b_copy(t).start()

    phi = _dotnn(r_ref[...], We_ref[...]) + be_ref[...]
    i_col = pair_idx_ref[:, 0:1]
    j_col = pair_idx_ref[:, 1:2]
    iota = jax.lax.broadcasted_iota(jnp.int32, (P, M), 1)
    oh_i = (i_col == iota).astype(jnp.float32)
    oh_j = (j_col == iota).astype(jnp.float32)
    phi_i = _dotnn(oh_i, phi)
    phi_j = _dotnn(oh_j, phi)
    sym_sum = phi_i + phi_j
    sym_abs = jnp.abs(phi_i - phi_j)
    sym_prod = phi_i * phi_j
    ap = (_dotnn(sym_sum, W1s_ref[...]) + _dotnn(sym_abs, W1a_ref[...])
          + _dotnn(sym_prod, W1m_ref[...]) + _dotnn(relf_ref[...], W1r_ref[...]))
    ap_bf = ap.astype(bf)
    W2_bf = W2_ref[...].astype(bf)
    W3_bf = W3_ref[...].astype(bf)

    for bt in range(B // TB):
        sl = pl.ds(bt * TB, TB)
        qg = jax.nn.gelu(_dotnn(q_ref[sl, :], Wq1_ref[...]) + bq1_ref[...])
        g = _dotnn(qg, Wq2_ref[...]) + bq2_ref[...]
        u_s[sl, :] = _dotnt(_dotnn(g, Wu_ref[...]), phi)
        az = _dotnn(g, W1z_ref[...]) + b1_ref[...]
        az_bf = az.astype(bf)
        h1 = _gelu_erf(az_bf[:, None, :] + ap_bf[None, :, :])
        h1 = h1.reshape(TB * P, H)
        h2pre = _dotnn(h1, W2_bf) + b2_ref[...]
        h2 = _gelu_erf(h2pre.astype(bf))
        v = _dotnn(h2, W3_bf) + b3_ref[...]
        v_s[sl, :] = v.reshape(TB, P)

    u_bf = u_s[...].astype(bf)
    v_bf = v_s[...].astype(bf)

    for t in range(NT):
        slot = t % NBUF
        a_copy(t).wait()
        b_copy(t).wait()
        s = (_dotnt(u_bf, A_buf[slot].astype(bf))
             + _dotnt(v_bf, Bp_buf[slot].astype(bf)))
        s = s - LAM * len_ref[:, pl.ds(t * TPROG, TPROG)]
        oslot = t % 2
        if t >= 2:
            o_copy(t - 2).wait()
        out_buf[oslot] = s
        o_copy(t).start()
        if t + NBUF < NT:
            a_copy(t + NBUF).start()
            b_copy(t + NBUF).start()
    for t in range(max(NT - 2, 0), NT):
        o_copy(t).wait()


@jax.jit
def kernel(q, r, A, B_pair, lengths, pair_index, relation_features,
           Wq1, bq1, Wq2, bq2, We, be, Wu, W1, b1, W2, b2, W3, b3):
    f32 = jnp.float32
    W1z = W1[:D_Z]
    W1s = W1[D_Z:D_Z + D_PHI]
    W1a = W1[D_Z + D_PHI:D_Z + 2 * D_PHI]
    W1m = W1[D_Z + 2 * D_PHI:D_Z + 3 * D_PHI]
    W1r = W1[D_Z + 3 * D_PHI:]
    bq1_2 = bq1.reshape(1, -1)
    bq2_2 = bq2.reshape(1, -1)
    be_2 = be.reshape(1, -1)
    b1_2 = b1.reshape(1, -1)
    b2_2 = b2.reshape(1, -1)
    b3_2 = b3.reshape(1, -1)
    pair_idx = pair_index.astype(jnp.int32)
    len_2 = lengths.reshape(1, NPROG)

    vmem = lambda: pl.BlockSpec(memory_space=pltpu.MemorySpace.VMEM)
    hbm = lambda: pl.BlockSpec(memory_space=pltpu.MemorySpace.HBM)
    scores = pl.pallas_call(
        _router_kernel,
        in_specs=[vmem() for _ in range(22)] + [hbm(), hbm()],
        out_specs=pl.BlockSpec(memory_space=pltpu.MemorySpace.HBM),
        out_shape=jax.ShapeDtypeStruct((B, NPROG), f32),
        scratch_shapes=[
            pltpu.VMEM((B, M), f32),
            pltpu.VMEM((B, P), f32),
            pltpu.VMEM((NBUF, TPROG, M), f32),
            pltpu.VMEM((NBUF, TPROG, P), f32),
            pltpu.VMEM((2, B, TPROG), f32),
            pltpu.SemaphoreType.DMA((NBUF,)),
            pltpu.SemaphoreType.DMA((2,)),
        ],
    )(q, r, pair_idx, relation_features,
      Wq1, bq1_2, Wq2, bq2_2, We, be_2, Wu,
      W1z, W1s, W1a, W1m, W1r, b1_2, W2, b2_2, W3, b3_2, len_2,
      A, B_pair)
    return scores


# speedup vs baseline: 1.1227x; 1.1227x over previous
import jax
import jax.numpy as jnp
from jax.experimental import pallas as pl
from jax.experimental.pallas import tpu as pltpu

B = 512
D_Q = 1024
D_Z = 512
D_RIN = 256
D_PHI = 128
D_R = 9
M = 512
P = 256
NPROG = 8192
LAM = 0.1
H = 96

TB = 256
TPROG = 2048
NBT = B // TB
NPT = NPROG // TPROG


def _gelu_erf(x):
    halfx = x * jnp.asarray(0.5, x.dtype)
    return halfx * jax.lax.erf(x * jnp.asarray(0.7071067811865476, x.dtype)) + halfx


def _dotnn(a, b):
    return jax.lax.dot_general(a, b, (((1,), (0,)), ((), ())),
                               preferred_element_type=jnp.float32)


def _dotnt(a, b):
    return jax.lax.dot_general(a, b, (((1,), (1,)), ((), ())),
                               preferred_element_type=jnp.float32)


def _fused_kernel(q_ref, r_ref, pair_idx_ref, relf_ref,
                  Wq1_ref, bq1_ref, Wq2_ref, bq2_ref,
                  We_ref, be_ref, Wu_ref,
                  W1z_ref, W1s_ref, W1a_ref, W1m_ref, W1r_ref, b1_ref,
                  W2_ref, b2_ref, W3_ref, b3_ref,
                  A_ref, Bp_ref, len_ref,
                  out_ref, u_s, v_s):
    step = pl.program_id(0)
    bf = jnp.bfloat16

    @pl.when(step < NBT)
    def _stage1():
        qg = jax.nn.gelu(_dotnn(q_ref[...], Wq1_ref[...]) + bq1_ref[...])
        g = _dotnn(qg, Wq2_ref[...]) + bq2_ref[...]

        phi = _dotnn(r_ref[...], We_ref[...]) + be_ref[...]
        row = pl.ds(step * TB, TB)
        u_s[row, :] = _dotnt(_dotnn(g, Wu_ref[...]), phi)

        i_col = pair_idx_ref[:, 0:1]
        j_col = pair_idx_ref[:, 1:2]
        iota = jax.lax.broadcasted_iota(jnp.int32, (P, M), 1)
        oh_i = (i_col == iota).astype(jnp.float32)
        oh_j = (j_col == iota).astype(jnp.float32)
        phi_i = _dotnn(oh_i, phi)
        phi_j = _dotnn(oh_j, phi)
        sym_sum = phi_i + phi_j
        sym_abs = jnp.abs(phi_i - phi_j)
        sym_prod = phi_i * phi_j

        az = _dotnn(g, W1z_ref[...]) + b1_ref[...]
        ap = (_dotnn(sym_sum, W1s_ref[...]) + _dotnn(sym_abs, W1a_ref[...])
              + _dotnn(sym_prod, W1m_ref[...]) + _dotnn(relf_ref[...], W1r_ref[...]))

        az_bf = az.astype(bf)
        ap_bf = ap.astype(bf)
        h1 = _gelu_erf(az_bf[:, None, :] + ap_bf[None, :, :])
        h1 = h1.reshape(TB * P, H)
        h2pre = _dotnn(h1, W2_ref[...].astype(bf)) + b2_ref[...]
        h2 = _gelu_erf(h2pre.astype(bf))
        v = _dotnn(h2, W3_ref[...].astype(bf)) + b3_ref[...]
        v_s[row, :] = v.reshape(TB, P)

    @pl.when(step >= NBT)
    def _stage2():
        s = (_dotnt(u_s[...].astype(bf), A_ref[...].astype(bf))
             + _dotnt(v_s[...].astype(bf), Bp_ref[...].astype(bf)))
        out_ref[...] = s - LAM * len_ref[...]


@jax.jit
def kernel(q, r, A, B_pair, lengths, pair_index, relation_features,
           Wq1, bq1, Wq2, bq2, We, be, Wu, W1, b1, W2, b2, W3, b3):
    f32 = jnp.float32
    W1z = W1[:D_Z]
    W1s = W1[D_Z:D_Z + D_PHI]
    W1a = W1[D_Z + D_PHI:D_Z + 2 * D_PHI]
    W1m = W1[D_Z + 2 * D_PHI:D_Z + 3 * D_PHI]
    W1r = W1[D_Z + 3 * D_PHI:]
    bq1_2 = bq1.reshape(1, -1)
    bq2_2 = bq2.reshape(1, -1)
    be_2 = be.reshape(1, -1)
    b1_2 = b1.reshape(1, -1)
    b2_2 = b2.reshape(1, -1)
    b3_2 = b3.reshape(1, -1)
    pair_idx = pair_index.astype(jnp.int32)
    len_2 = lengths.reshape(1, NPROG)

    rep = lambda shape: pl.BlockSpec(shape, lambda s: (0,) * len(shape))
    qmap = lambda s: (jnp.minimum(s, NBT - 1), 0)
    pmap = lambda s: (jnp.maximum(s - NBT, 0), 0)
    cmap = lambda s: (0, jnp.maximum(s - NBT, 0))
    scores = pl.pallas_call(
        _fused_kernel,
        grid=(NBT + NPT,),
        in_specs=[
            pl.BlockSpec((TB, D_Q), qmap),
            rep((M, D_RIN)),
            rep((P, 2)),
            rep((P, D_R)),
            rep((D_Q, 512)), rep((1, 512)),
            rep((512, D_Z)), rep((1, D_Z)),
            rep((D_RIN, D_PHI)), rep((1, D_PHI)),
            rep((D_Z, D_PHI)),
            rep((D_Z, H)), rep((D_PHI, H)), rep((D_PHI, H)), rep((D_PHI, H)),
            rep((D_R, H)), rep((1, H)),
            rep((H, H)), rep((1, H)),
            rep((H, 1)), rep((1, 1)),
            pl.BlockSpec((TPROG, M), pmap),
            pl.BlockSpec((TPROG, P), pmap),
            pl.BlockSpec((1, TPROG), cmap),
        ],
        out_specs=pl.BlockSpec((B, TPROG), cmap),
        out_shape=jax.ShapeDtypeStruct((B, NPROG), f32),
        scratch_shapes=[
            pltpu.VMEM((B, M), f32),
            pltpu.VMEM((B, P), f32),
        ],
    )(q, r, pair_idx, relation_features,
      Wq1, bq1_2, Wq2, bq2_2, We, be_2, Wu,
      W1z, W1s, W1a, W1m, W1r, b1_2, W2, b2_2, W3, b3_2,
      A, B_pair, len_2)
    return scores
